# baseline (speedup 1.0000x reference)
import functools

import jax
import jax.numpy as jnp
from jax import lax
from jax.experimental import pallas as pl
from jax.experimental.pallas import tpu as pltpu
from jax.experimental.pallas import tpu_sc as plsc

HIGH = lax.Precision.HIGHEST

N = 10000
E = 320000
A = E + N
NW = 32
EW = 10320
A_PAD = EW * NW
NP = 10240
ROWS_W = NP // 16

_MESH = None


def _mesh():
    global _MESH
    if _MESH is None:
        _MESH = plsc.VectorSubcoreMesh(core_axis_name="c", subcore_axis_name="s")
    return _MESH


def _proj_body(x_ref, wl_ref, bl_ref, wr_ref, br_ref, absa_ref,
               xl_ref, xr_ref, p_ref):
    xb = x_ref[...]
    xl = jnp.dot(xb, wl_ref[...], precision=HIGH) + bl_ref[...]
    xr = jnp.dot(xb, wr_ref[...], precision=HIGH) + br_ref[...]
    xl_ref[...] = xl
    xr_ref[...] = xr
    p_ref[...] = jnp.dot(jnp.abs(xl), absa_ref[...], precision=HIGH)


def _proj(x, wl, bl, wr, br, absa, bn=1000):
    n, d = x.shape
    f = wl.shape[1]
    h = absa.shape[1]
    return pl.pallas_call(
        _proj_body,
        grid=(n // bn,),
        in_specs=[
            pl.BlockSpec((bn, d), lambda i: (i, 0)),
            pl.BlockSpec((d, f), lambda i: (0, 0)),
            pl.BlockSpec((1, f), lambda i: (0, 0)),
            pl.BlockSpec((d, f), lambda i: (0, 0)),
            pl.BlockSpec((1, f), lambda i: (0, 0)),
            pl.BlockSpec((f, h), lambda i: (0, 0)),
        ],
        out_specs=[
            pl.BlockSpec((bn, f), lambda i: (i, 0)),
            pl.BlockSpec((bn, f), lambda i: (i, 0)),
            pl.BlockSpec((bn, h), lambda i: (i, 0)),
        ],
        out_shape=[
            jax.ShapeDtypeStruct((n, f), jnp.float32),
            jax.ShapeDtypeStruct((n, f), jnp.float32),
            jax.ShapeDtypeStruct((n, h), jnp.float32),
        ],
    )(x, wl, bl, wr, br, absa)


def _colmax_body(p_ref, out_ref):
    out_ref[...] = jnp.max(p_ref[...], axis=0, keepdims=True)


def _colmax(p):
    _, h = p.shape
    return pl.pallas_call(
        _colmax_body,
        out_shape=jax.ShapeDtypeStruct((1, h), jnp.float32),
    )(p)


def _edge_body(gxl_ref, gxr_ref, maxp_ref, amask_ref, absa_ref, emaskt_ref,
               pick_ref, uw_ref, *, h, be, feff, fext):
    a = gxl_ref[...]
    b = gxr_ref[...]
    z = a + b
    lz = jnp.maximum(z, 0.2 * z)
    logits = jnp.dot(lz, amask_ref[...], precision=HIGH)
    sh = jnp.dot(jnp.abs(b), absa_ref[...], precision=HIGH) + maxp_ref[...]
    lshw = jnp.dot(logits - sh, emaskt_ref[...], precision=HIGH)
    exw = jnp.exp(lshw)
    eid = pl.program_id(0) * be + lax.broadcasted_iota(jnp.int32, (be, 1), 0)
    exw = jnp.where(eid < A, exw, 0.0)
    ex = jnp.dot(exw, pick_ref[...], precision=HIGH)
    pad = fext - feff - h
    uw_ref[...] = jnp.concatenate(
        [a[:, :feff] * exw, ex, jnp.zeros((be, pad), jnp.float32)], axis=1)


def _edge(gxl, gxr, maxp, amask, absa, emaskt, pick, fext, be=512):
    a_pad, f = gxl.shape
    h = amask.shape[1]
    feff = emaskt.shape[1]
    return pl.pallas_call(
        functools.partial(_edge_body, h=h, be=be, feff=feff, fext=fext),
        grid=(a_pad // be,),
        in_specs=[
            pl.BlockSpec((be, f), lambda i: (i, 0)),
            pl.BlockSpec((be, f), lambda i: (i, 0)),
            pl.BlockSpec((1, h), lambda i: (0, 0)),
            pl.BlockSpec((f, h), lambda i: (0, 0)),
            pl.BlockSpec((f, h), lambda i: (0, 0)),
            pl.BlockSpec((h, feff), lambda i: (0, 0)),
            pl.BlockSpec((feff, h), lambda i: (0, 0)),
        ],
        out_specs=pl.BlockSpec((be, fext), lambda i: (i, 0)),
        out_shape=jax.ShapeDtypeStruct((a_pad, fext), jnp.float32),
    )(gxl, gxr, maxp, amask, absa, emaskt, pick)


def _mid_body(uo_ref, b1_ref, emaskt_ref, wl_ref, bl_ref,
              wr_ref, br_ref, absa_ref, xl_ref, xr_ref, p_ref):
    u = uo_ref[0, :, 0:512] + uo_ref[1, :, 0:512]
    d8 = uo_ref[0, :, 512:520] + uo_ref[1, :, 512:520]
    db = jnp.dot(d8, emaskt_ref[...], precision=HIGH) + 1e-16
    hid = u / db + b1_ref[...]
    hid = jnp.where(hid > 0, hid, jnp.exp(jnp.minimum(hid, 0.0)) - 1.0)
    xl = jnp.dot(hid, wl_ref[...], precision=HIGH) + bl_ref[...]
    xr = jnp.dot(hid, wr_ref[...], precision=HIGH) + br_ref[...]
    pad = jnp.zeros((xl.shape[0], 128 - xl.shape[1]), jnp.float32)
    xl_ref[...] = jnp.concatenate([xl, pad], axis=1)
    xr_ref[...] = jnp.concatenate([xr, pad], axis=1)
    p_ref[...] = jnp.dot(jnp.abs(xl), absa_ref[...], precision=HIGH)


def _mid(uo, b1, emaskt, wl2, bl2, wr2, br2, absa2, bn=1024):
    f = uo.shape[2]
    f2 = wl2.shape[1]
    h2 = absa2.shape[1]
    return pl.pallas_call(
        _mid_body,
        grid=(NP // bn,),
        in_specs=[
            pl.BlockSpec((2, bn, f), lambda i: (0, i, 0)),
            pl.BlockSpec((1, 512), lambda i: (0, 0)),
            pl.BlockSpec((8, 512), lambda i: (0, 0)),
            pl.BlockSpec((512, f2), lambda i: (0, 0)),
            pl.BlockSpec((1, f2), lambda i: (0, 0)),
            pl.BlockSpec((512, f2), lambda i: (0, 0)),
            pl.BlockSpec((1, f2), lambda i: (0, 0)),
            pl.BlockSpec((f2, h2), lambda i: (0, 0)),
        ],
        out_specs=[
            pl.BlockSpec((bn, 128), lambda i: (i, 0)),
            pl.BlockSpec((bn, 128), lambda i: (i, 0)),
            pl.BlockSpec((bn, h2), lambda i: (i, 0)),
        ],
        out_shape=[
            jax.ShapeDtypeStruct((NP, 128), jnp.float32),
            jax.ShapeDtypeStruct((NP, 128), jnp.float32),
            jax.ShapeDtypeStruct((NP, h2), jnp.float32),
        ],
    )(uo, b1, emaskt, wl2, bl2, wr2, br2, absa2)


def _fin_body(uo_ref, b2_ref, wlin_ref, blin_ref, out_ref):
    u = uo_ref[0, :, 0:64] + uo_ref[1, :, 0:64]
    d = uo_ref[0, :, 64:65] + uo_ref[1, :, 64:65]
    hid = u / (d + 1e-16) + b2_ref[...]
    hid = jnp.where(hid > 0, hid, jnp.exp(jnp.minimum(hid, 0.0)) - 1.0)
    out_ref[...] = jnp.dot(hid, wlin_ref[...], precision=HIGH) + blin_ref[...]


def _fin(uo, b2, wlin, blin, bn=1024):
    f = uo.shape[2]
    fo = wlin.shape[1]
    return pl.pallas_call(
        _fin_body,
        grid=(NP // bn,),
        in_specs=[
            pl.BlockSpec((2, bn, f), lambda i: (0, i, 0)),
            pl.BlockSpec((1, 64), lambda i: (0, 0)),
            pl.BlockSpec((64, fo), lambda i: (0, 0)),
            pl.BlockSpec((1, fo), lambda i: (0, 0)),
        ],
        out_specs=pl.BlockSpec((bn, fo), lambda i: (i, 0)),
        out_shape=jax.ShapeDtypeStruct((NP, fo), jnp.float32),
    )(uo, b2, wlin, blin)


def _gather(xl, xr, src, dst, c2=80):
    f = xl.shape[1]
    nch = EW // c2

    @functools.partial(
        pl.kernel,
        mesh=_mesh(),
        out_type=[
            jax.ShapeDtypeStruct((A_PAD, f), jnp.float32),
            jax.ShapeDtypeStruct((A_PAD, f), jnp.float32),
        ],
        scratch_types=[
            pltpu.VMEM((c2,), jnp.int32),
            pltpu.VMEM((c2,), jnp.int32),
            pltpu.VMEM((c2, f), jnp.float32),
            pltpu.VMEM((c2, f), jnp.float32),
            pltpu.SemaphoreType.DMA,
            pltpu.SemaphoreType.DMA,
        ],
    )
    def k(xl_h, xr_h, src_h, dst_h, gxl_h, gxr_h,
          sidx, didx, bxl, bxr, sem0, sem1):
        wid = lax.axis_index("s") * 2 + lax.axis_index("c")
        base = wid * EW

        @pl.loop(0, nch)
        def _(i):
            off = base + i * c2
            pltpu.sync_copy(src_h.at[pl.ds(off, c2)], sidx)
            pltpu.sync_copy(dst_h.at[pl.ds(off, c2)], didx)
            d0 = pltpu.async_copy(xl_h.at[sidx], bxl, sem0)
            d1 = pltpu.async_copy(xr_h.at[didx], bxr, sem1)
            d0.wait()
            d1.wait()
            pltpu.sync_copy(bxl, gxl_h.at[pl.ds(off, c2)])
            pltpu.sync_copy(bxr, gxr_h.at[pl.ds(off, c2)])

    return k(xl, xr, src, dst)


def _segsum(uw, dst, rowidx, z80, slices, c4=80):
    f = uw.shape[1]
    cw = f // slices
    nch = EW // c4
    nzch = ROWS_W // c4

    @functools.partial(
        pl.kernel,
        mesh=_mesh(),
        out_type=jax.ShapeDtypeStruct((2, NP, f), jnp.float32),
        scratch_types=[
            pltpu.VMEM_SHARED((NP, cw), jnp.float32),
            pltpu.VMEM((c4, cw), jnp.float32),
            pltpu.VMEM((c4,), jnp.int32),
            pltpu.VMEM((c4,), jnp.int32),
        ],
    )
    def k(uw_h, dst_h, ri_h, z80_h, uo_h, slabm, bufm, ibuf, ribuf):
        c = lax.axis_index("c")
        s = lax.axis_index("s")
        wid = s * 2 + c
        base = wid * EW
        rs = s * ROWS_W

        for sl in range(slices):
            pltpu.sync_copy(z80_h, bufm)

            @pl.loop(0, nzch)
            def _(r):
                pltpu.sync_copy(ri_h.at[pl.ds(rs + r * c4, c4)], ribuf)
                pltpu.sync_copy(bufm, slabm.at[ribuf])

            plsc.subcore_barrier()

            @pl.loop(0, nch)
            def _(i):
                off = base + i * c4
                pltpu.sync_copy(dst_h.at[pl.ds(off, c4)], ibuf)
                if slices == 1:
                    pltpu.sync_copy(uw_h.at[pl.ds(off, c4)], bufm)
                else:
                    pltpu.sync_copy(
                        uw_h.at[pl.ds(off, c4), pl.ds(sl * cw, cw)], bufm)
                pltpu.sync_copy(bufm, slabm.at[ibuf], add=True)

            plsc.subcore_barrier()

            @pl.loop(0, nzch)
            def _(r):
                row = rs + r * c4
                pltpu.sync_copy(ri_h.at[pl.ds(row, c4)], ribuf)
                pltpu.sync_copy(slabm.at[ribuf], bufm)
                if slices == 1:
                    pltpu.sync_copy(bufm, uo_h.at[c, pl.ds(row, c4)])
                else:
                    pltpu.sync_copy(
                        bufm, uo_h.at[c, pl.ds(row, c4), pl.ds(sl * cw, cw)])

            plsc.subcore_barrier()

    return k(uw, dst, rowidx, z80)


def kernel(x, edge_index, Wl1, bl1, Wr1, br1, att1, b1,
           Wl2, bl2, Wr2, br2, att2, b2, Wlin, blin):
    f32 = jnp.float32
    loop = jnp.arange(N, dtype=jnp.int32)
    padz = jnp.zeros((A_PAD - A,), jnp.int32)
    src = jnp.concatenate([edge_index[0].astype(jnp.int32), loop, padz])
    dst = jnp.concatenate([edge_index[1].astype(jnp.int32), loop, padz])
    rowidx = jnp.arange(NP, dtype=jnp.int32)

    attf1 = att1.reshape(-1).astype(f32)
    hm1 = (jnp.arange(512)[:, None] // 64) == jnp.arange(8)[None, :]
    amask1 = jnp.where(hm1, attf1[:, None], 0.0)
    absa1 = jnp.where(hm1, jnp.abs(attf1)[:, None], 0.0)
    emaskt1 = hm1.astype(f32).T
    attf2 = att2.reshape(-1).astype(f32)
    amask2 = jnp.concatenate(
        [attf2[:, None], jnp.zeros((64, 1), f32)], axis=0)
    absa2 = jnp.abs(amask2)
    absa2u = jnp.abs(attf2)[:, None]
    emaskt2 = jnp.ones((1, 64), f32)

    pick1 = (jnp.arange(512)[:, None] == jnp.arange(8)[None, :] * 64
             ).astype(f32)
    pick2 = (jnp.arange(64)[:, None] == 0).astype(f32)

    z80 = jnp.zeros((80, 128), f32)

    xl1, xr1, p1 = _proj(x, Wl1, bl1.reshape(1, -1), Wr1,
                         br1.reshape(1, -1), absa1)
    maxp1 = _colmax(p1)
    gxl1, gxr1 = _gather(xl1, xr1, src, dst)
    uw1 = _edge(gxl1, gxr1, maxp1, amask1, absa1, emaskt1, pick1, fext=640)
    uo1 = _segsum(uw1, dst, rowidx, z80, slices=5)

    xl2, xr2, p2 = _mid(uo1, b1.reshape(1, -1), emaskt1,
                        Wl2, bl2.reshape(1, -1), Wr2,
                        br2.reshape(1, -1), absa2u)
    maxp2 = _colmax(p2)
    gxl2, gxr2 = _gather(xl2, xr2, src, dst)
    uw2 = _edge(gxl2, gxr2, maxp2, amask2, absa2, emaskt2, pick2, fext=128)
    uo2 = _segsum(uw2, dst, rowidx, z80, slices=1)

    out = _fin(uo2, b2.reshape(1, -1), Wlin, blin.reshape(1, -1))
    return out[:N]


# speedup vs baseline: 1.3328x; 1.3328x over previous
import functools

import jax
import jax.numpy as jnp
from jax import lax
from jax.experimental import pallas as pl
from jax.experimental.pallas import tpu as pltpu
from jax.experimental.pallas import tpu_sc as plsc

HIGH = lax.Precision.HIGHEST

N = 10000
E = 320000
A = E + N
NW = 32
EW = 10320
A_PAD = EW * NW
NP = 10240
ROWS_W = NP // 16

_MESH = None


def _mesh():
    global _MESH
    if _MESH is None:
        _MESH = plsc.VectorSubcoreMesh(core_axis_name="c", subcore_axis_name="s")
    return _MESH


def _proj_body(x_ref, wl_ref, bl_ref, wr_ref, br_ref, absa_ref,
               xl_ref, xr_ref, p_ref):
    xb = x_ref[...]
    xl = jnp.dot(xb, wl_ref[...], precision=HIGH) + bl_ref[...]
    xr = jnp.dot(xb, wr_ref[...], precision=HIGH) + br_ref[...]
    xl_ref[...] = xl
    xr_ref[...] = xr
    p_ref[...] = jnp.dot(jnp.abs(xl), absa_ref[...], precision=HIGH)


def _proj(x, wl, bl, wr, br, absa, bn=1000):
    n, d = x.shape
    f = wl.shape[1]
    h = absa.shape[1]
    return pl.pallas_call(
        _proj_body,
        grid=(n // bn,),
        in_specs=[
            pl.BlockSpec((bn, d), lambda i: (i, 0)),
            pl.BlockSpec((d, f), lambda i: (0, 0)),
            pl.BlockSpec((1, f), lambda i: (0, 0)),
            pl.BlockSpec((d, f), lambda i: (0, 0)),
            pl.BlockSpec((1, f), lambda i: (0, 0)),
            pl.BlockSpec((f, h), lambda i: (0, 0)),
        ],
        out_specs=[
            pl.BlockSpec((bn, f), lambda i: (i, 0)),
            pl.BlockSpec((bn, f), lambda i: (i, 0)),
            pl.BlockSpec((bn, h), lambda i: (i, 0)),
        ],
        out_shape=[
            jax.ShapeDtypeStruct((n, f), jnp.float32),
            jax.ShapeDtypeStruct((n, f), jnp.float32),
            jax.ShapeDtypeStruct((n, h), jnp.float32),
        ],
    )(x, wl, bl, wr, br, absa)


def _colmax_body(p_ref, out_ref):
    out_ref[...] = jnp.max(p_ref[...], axis=0, keepdims=True)


def _colmax(p):
    _, h = p.shape
    return pl.pallas_call(
        _colmax_body,
        out_shape=jax.ShapeDtypeStruct((1, h), jnp.float32),
    )(p)


def _edge_body(gxl_ref, gxr_ref, maxp_ref, amask_ref, absa_ref, emaskt_ref,
               pick_ref, uw_ref, *, h, be, feff, fext):
    a = gxl_ref[...]
    b = gxr_ref[...]
    z = a + b
    lz = jnp.maximum(z, 0.2 * z)
    logits = jnp.dot(lz, amask_ref[...], precision=HIGH)
    sh = jnp.dot(jnp.abs(b), absa_ref[...], precision=HIGH) + maxp_ref[...]
    ex = jnp.exp(logits - sh)
    eid = pl.program_id(0) * be + lax.broadcasted_iota(jnp.int32, (be, 1), 0)
    ex = jnp.where(eid < A, ex, 0.0)
    exb = jnp.dot(ex, emaskt_ref[...], precision=HIGH)
    pad = fext - feff - h
    uw_ref[...] = jnp.concatenate(
        [a[:, :feff] * exb, ex, jnp.zeros((be, pad), jnp.float32)], axis=1)


def _edge(gxl, gxr, maxp, amask, absa, emaskt, pick, fext, be=512):
    a_pad, f = gxl.shape
    h = amask.shape[1]
    feff = emaskt.shape[1]
    return pl.pallas_call(
        functools.partial(_edge_body, h=h, be=be, feff=feff, fext=fext),
        grid=(a_pad // be,),
        in_specs=[
            pl.BlockSpec((be, f), lambda i: (i, 0)),
            pl.BlockSpec((be, f), lambda i: (i, 0)),
            pl.BlockSpec((1, h), lambda i: (0, 0)),
            pl.BlockSpec((f, h), lambda i: (0, 0)),
            pl.BlockSpec((f, h), lambda i: (0, 0)),
            pl.BlockSpec((h, feff), lambda i: (0, 0)),
            pl.BlockSpec((feff, h), lambda i: (0, 0)),
        ],
        out_specs=pl.BlockSpec((be, fext), lambda i: (i, 0)),
        out_shape=jax.ShapeDtypeStruct((a_pad, fext), jnp.float32),
    )(gxl, gxr, maxp, amask, absa, emaskt, pick)


def _mid_body(uo_ref, b1_ref, emaskt_ref, wl_ref, bl_ref,
              wr_ref, br_ref, absa_ref, xl_ref, xr_ref, p_ref):
    u = uo_ref[0, :, 0:512] + uo_ref[1, :, 0:512]
    d8 = uo_ref[0, :, 512:520] + uo_ref[1, :, 512:520]
    db = jnp.dot(d8, emaskt_ref[...], precision=HIGH) + 1e-16
    hid = u / db + b1_ref[...]
    hid = jnp.where(hid > 0, hid, jnp.exp(jnp.minimum(hid, 0.0)) - 1.0)
    xl = jnp.dot(hid, wl_ref[...], precision=HIGH) + bl_ref[...]
    xr = jnp.dot(hid, wr_ref[...], precision=HIGH) + br_ref[...]
    pad = jnp.zeros((xl.shape[0], 128 - xl.shape[1]), jnp.float32)
    xl_ref[...] = jnp.concatenate([xl, pad], axis=1)
    xr_ref[...] = jnp.concatenate([xr, pad], axis=1)
    p_ref[...] = jnp.dot(jnp.abs(xl), absa_ref[...], precision=HIGH)


def _mid(uo, b1, emaskt, wl2, bl2, wr2, br2, absa2, bn=1024):
    f = uo.shape[2]
    f2 = wl2.shape[1]
    h2 = absa2.shape[1]
    return pl.pallas_call(
        _mid_body,
        grid=(NP // bn,),
        in_specs=[
            pl.BlockSpec((2, bn, f), lambda i: (0, i, 0)),
            pl.BlockSpec((1, 512), lambda i: (0, 0)),
            pl.BlockSpec((8, 512), lambda i: (0, 0)),
            pl.BlockSpec((512, f2), lambda i: (0, 0)),
            pl.BlockSpec((1, f2), lambda i: (0, 0)),
            pl.BlockSpec((512, f2), lambda i: (0, 0)),
            pl.BlockSpec((1, f2), lambda i: (0, 0)),
            pl.BlockSpec((f2, h2), lambda i: (0, 0)),
        ],
        out_specs=[
            pl.BlockSpec((bn, 128), lambda i: (i, 0)),
            pl.BlockSpec((bn, 128), lambda i: (i, 0)),
            pl.BlockSpec((bn, h2), lambda i: (i, 0)),
        ],
        out_shape=[
            jax.ShapeDtypeStruct((NP, 128), jnp.float32),
            jax.ShapeDtypeStruct((NP, 128), jnp.float32),
            jax.ShapeDtypeStruct((NP, h2), jnp.float32),
        ],
    )(uo, b1, emaskt, wl2, bl2, wr2, br2, absa2)


def _fin_body(uo_ref, b2_ref, wlin_ref, blin_ref, out_ref):
    u = uo_ref[0, :, 0:64] + uo_ref[1, :, 0:64]
    d = uo_ref[0, :, 64:65] + uo_ref[1, :, 64:65]
    hid = u / (d + 1e-16) + b2_ref[...]
    hid = jnp.where(hid > 0, hid, jnp.exp(jnp.minimum(hid, 0.0)) - 1.0)
    out_ref[...] = jnp.dot(hid, wlin_ref[...], precision=HIGH) + blin_ref[...]


def _fin(uo, b2, wlin, blin, bn=1024):
    f = uo.shape[2]
    fo = wlin.shape[1]
    return pl.pallas_call(
        _fin_body,
        grid=(NP // bn,),
        in_specs=[
            pl.BlockSpec((2, bn, f), lambda i: (0, i, 0)),
            pl.BlockSpec((1, 64), lambda i: (0, 0)),
            pl.BlockSpec((64, fo), lambda i: (0, 0)),
            pl.BlockSpec((1, fo), lambda i: (0, 0)),
        ],
        out_specs=pl.BlockSpec((bn, fo), lambda i: (i, 0)),
        out_shape=jax.ShapeDtypeStruct((NP, fo), jnp.float32),
    )(uo, b2, wlin, blin)


def _gather(xl, xr, src, dst, c2=40):
    f = xl.shape[1]
    nch = EW // c2
    npairs = nch // 2

    @functools.partial(
        pl.kernel,
        mesh=_mesh(),
        out_type=[
            jax.ShapeDtypeStruct((A_PAD, f), jnp.float32),
            jax.ShapeDtypeStruct((A_PAD, f), jnp.float32),
        ],
        scratch_types=[
            pltpu.VMEM((c2,), jnp.int32),
            pltpu.VMEM((c2,), jnp.int32),
            pltpu.VMEM((c2,), jnp.int32),
            pltpu.VMEM((c2,), jnp.int32),
            pltpu.VMEM((c2, f), jnp.float32),
            pltpu.VMEM((c2, f), jnp.float32),
            pltpu.VMEM((c2, f), jnp.float32),
            pltpu.VMEM((c2, f), jnp.float32),
        ] + [pltpu.SemaphoreType.DMA] * 8,
    )
    def k(xl_h, xr_h, src_h, dst_h, gxl_h, gxr_h,
          si0, di0, si1, di1, bl0, br0, bl1, br1,
          sgl0, sgr0, sgl1, sgr1, swl0, swr0, swl1, swr1):
        wid = lax.axis_index("s") * 2 + lax.axis_index("c")
        base = wid * EW

        def wait_g(bl, br, sgl, sgr):
            pltpu.make_async_copy(xl_h.at[pl.ds(0, c2)], bl, sgl).wait()
            pltpu.make_async_copy(xr_h.at[pl.ds(0, c2)], br, sgr).wait()

        def wait_w(bl, br, swl, swr):
            pltpu.make_async_copy(bl, gxl_h.at[pl.ds(0, c2)], swl).wait()
            pltpu.make_async_copy(br, gxr_h.at[pl.ds(0, c2)], swr).wait()

        pltpu.sync_copy(src_h.at[pl.ds(base, c2)], si0)
        pltpu.sync_copy(dst_h.at[pl.ds(base, c2)], di0)
        pltpu.async_copy(xl_h.at[si0], bl0, sgl0)
        pltpu.async_copy(xr_h.at[di0], br0, sgr0)

        @pl.loop(0, npairs)
        def _(p):
            c0 = base + (2 * p) * c2
            c1 = c0 + c2
            cn = c0 + 2 * c2
            wait_g(bl0, br0, sgl0, sgr0)
            pltpu.async_copy(bl0, gxl_h.at[pl.ds(c0, c2)], swl0)
            pltpu.async_copy(br0, gxr_h.at[pl.ds(c0, c2)], swr0)

            pltpu.sync_copy(src_h.at[pl.ds(c1, c2)], si1)
            pltpu.sync_copy(dst_h.at[pl.ds(c1, c2)], di1)

            @pl.when(p > 0)
            def _():
                wait_w(bl1, br1, swl1, swr1)

            pltpu.async_copy(xl_h.at[si1], bl1, sgl1)
            pltpu.async_copy(xr_h.at[di1], br1, sgr1)
            wait_g(bl1, br1, sgl1, sgr1)
            pltpu.async_copy(bl1, gxl_h.at[pl.ds(c1, c2)], swl1)
            pltpu.async_copy(br1, gxr_h.at[pl.ds(c1, c2)], swr1)

            @pl.when(p < npairs - 1)
            def _():
                pltpu.sync_copy(src_h.at[pl.ds(cn, c2)], si0)
                pltpu.sync_copy(dst_h.at[pl.ds(cn, c2)], di0)
                wait_w(bl0, br0, swl0, swr0)
                pltpu.async_copy(xl_h.at[si0], bl0, sgl0)
                pltpu.async_copy(xr_h.at[di0], br0, sgr0)

        wait_w(bl0, br0, swl0, swr0)
        wait_w(bl1, br1, swl1, swr1)

    return k(xl, xr, src, dst)


def _segsum(uw, dst, rowidx, z80, slices, c4=40):
    f = uw.shape[1]
    cw = f // slices
    nch = EW // c4
    nzch = ROWS_W // c4

    @functools.partial(
        pl.kernel,
        mesh=_mesh(),
        out_type=jax.ShapeDtypeStruct((2, NP, f), jnp.float32),
        scratch_types=[
            pltpu.VMEM_SHARED((NP, cw), jnp.float32),
            pltpu.VMEM((c4, cw), jnp.float32),
            pltpu.VMEM((c4, cw), jnp.float32),
            pltpu.VMEM((c4,), jnp.int32),
            pltpu.VMEM((c4,), jnp.int32),
            pltpu.VMEM((c4,), jnp.int32),
        ] + [pltpu.SemaphoreType.DMA] * 6,
    )
    def k(uw_h, dst_h, ri_h, z80_h, uo_h, slabm, bufm, bufm2, ibuf, ibuf2,
          ribuf, sli0, slm0, sli1, slm1, ssc0, ssc1):
        c = lax.axis_index("c")
        s = lax.axis_index("s")
        wid = s * 2 + c
        base = wid * EW
        rs = s * ROWS_W

        for sl in range(slices):
            pltpu.sync_copy(z80_h, bufm)

            @pl.loop(0, nzch)
            def _(r):
                pltpu.sync_copy(ri_h.at[pl.ds(rs + r * c4, c4)], ribuf)
                pltpu.sync_copy(bufm, slabm.at[ribuf])

            plsc.subcore_barrier()

            def load(i, ib, bm, sli, slm):
                off = base + i * c4
                pltpu.async_copy(dst_h.at[pl.ds(off, c4)], ib, sli)
                if slices == 1:
                    pltpu.async_copy(uw_h.at[pl.ds(off, c4)], bm, slm)
                else:
                    pltpu.async_copy(
                        uw_h.at[pl.ds(off, c4), pl.ds(sl * cw, cw)], bm, slm)

            def wait_load(ib, bm, sli, slm):
                pltpu.make_async_copy(dst_h.at[pl.ds(0, c4)], ib, sli).wait()
                pltpu.make_async_copy(
                    uw_h.at[pl.ds(0, c4), pl.ds(0, cw)], bm, slm).wait()

            def wait_sc(bm, ssc):
                pltpu.make_async_copy(bm, slabm.at[pl.ds(0, c4)], ssc).wait()

            load(0, ibuf, bufm, sli0, slm0)

            @pl.loop(0, nch // 2)
            def _(p):
                wait_load(ibuf, bufm, sli0, slm0)
                pltpu.async_copy(bufm, slabm.at[ibuf], ssc0, add=True)

                @pl.when(p > 0)
                def _():
                    wait_sc(bufm2, ssc1)

                load(2 * p + 1, ibuf2, bufm2, sli1, slm1)
                wait_load(ibuf2, bufm2, sli1, slm1)
                pltpu.async_copy(bufm2, slabm.at[ibuf2], ssc1, add=True)

                @pl.when(p < nch // 2 - 1)
                def _():
                    wait_sc(bufm, ssc0)
                    load(2 * p + 2, ibuf, bufm, sli0, slm0)

            wait_sc(bufm, ssc0)
            wait_sc(bufm2, ssc1)
            plsc.subcore_barrier()

            @pl.loop(0, nzch)
            def _(r):
                row = rs + r * c4
                pltpu.sync_copy(ri_h.at[pl.ds(row, c4)], ribuf)
                pltpu.sync_copy(slabm.at[ribuf], bufm)
                if slices == 1:
                    pltpu.sync_copy(bufm, uo_h.at[c, pl.ds(row, c4)])
                else:
                    pltpu.sync_copy(
                        bufm, uo_h.at[c, pl.ds(row, c4), pl.ds(sl * cw, cw)])

            plsc.subcore_barrier()

    return k(uw, dst, rowidx, z80)


def kernel(x, edge_index, Wl1, bl1, Wr1, br1, att1, b1,
           Wl2, bl2, Wr2, br2, att2, b2, Wlin, blin):
    f32 = jnp.float32
    loop = jnp.arange(N, dtype=jnp.int32)
    padz = jnp.zeros((A_PAD - A,), jnp.int32)
    src = jnp.concatenate([edge_index[0].astype(jnp.int32), loop, padz])
    dst = jnp.concatenate([edge_index[1].astype(jnp.int32), loop, padz])
    rowidx = jnp.arange(NP, dtype=jnp.int32)

    attf1 = att1.reshape(-1).astype(f32)
    hm1 = (jnp.arange(512)[:, None] // 64) == jnp.arange(8)[None, :]
    amask1 = jnp.where(hm1, attf1[:, None], 0.0)
    absa1 = jnp.where(hm1, jnp.abs(attf1)[:, None], 0.0)
    emaskt1 = hm1.astype(f32).T
    attf2 = att2.reshape(-1).astype(f32)
    amask2 = jnp.concatenate(
        [attf2[:, None], jnp.zeros((64, 1), f32)], axis=0)
    absa2 = jnp.abs(amask2)
    absa2u = jnp.abs(attf2)[:, None]
    emaskt2 = jnp.ones((1, 64), f32)

    pick1 = (jnp.arange(512)[:, None] == jnp.arange(8)[None, :] * 64
             ).astype(f32)
    pick2 = (jnp.arange(64)[:, None] == 0).astype(f32)

    z80 = jnp.zeros((40, 128), f32)

    xl1, xr1, p1 = _proj(x, Wl1, bl1.reshape(1, -1), Wr1,
                         br1.reshape(1, -1), absa1)
    maxp1 = _colmax(p1)
    gxl1, gxr1 = _gather(xl1, xr1, src, dst)
    uw1 = _edge(gxl1, gxr1, maxp1, amask1, absa1, emaskt1, pick1, fext=640)
    uo1 = _segsum(uw1, dst, rowidx, z80, slices=5)

    xl2, xr2, p2 = _mid(uo1, b1.reshape(1, -1), emaskt1,
                        Wl2, bl2.reshape(1, -1), Wr2,
                        br2.reshape(1, -1), absa2u)
    maxp2 = _colmax(p2)
    gxl2, gxr2 = _gather(xl2, xr2, src, dst)
    uw2 = _edge(gxl2, gxr2, maxp2, amask2, absa2, emaskt2, pick2, fext=128)
    uo2 = _segsum(uw2, dst, rowidx, z80, slices=1)

    out = _fin(uo2, b2.reshape(1, -1), Wlin, blin.reshape(1, -1))
    return out[:N]


# speedup vs baseline: 1.3685x; 1.0268x over previous
import functools

import jax
import jax.numpy as jnp
from jax import lax
from jax.experimental import pallas as pl
from jax.experimental.pallas import tpu as pltpu
from jax.experimental.pallas import tpu_sc as plsc

HIGH = lax.Precision.HIGHEST

N = 10000
E = 320000
A = E + N
NW = 32
EW = 10320
A_PAD = EW * NW
NP = 10240
ROWS_W = NP // 16

_MESH = None


def _mesh():
    global _MESH
    if _MESH is None:
        _MESH = plsc.VectorSubcoreMesh(core_axis_name="c", subcore_axis_name="s")
    return _MESH


def _proj_body(x_ref, wl_ref, bl_ref, wr_ref, br_ref, absa_ref,
               xl_ref, xr_ref, p_ref):
    xb = x_ref[...]
    xl = jnp.dot(xb, wl_ref[...], precision=HIGH) + bl_ref[...]
    xr = jnp.dot(xb, wr_ref[...], precision=HIGH) + br_ref[...]
    xl_ref[...] = xl
    xr_ref[...] = xr
    p_ref[...] = jnp.dot(jnp.abs(xl), absa_ref[...], precision=HIGH)


def _proj(x, wl, bl, wr, br, absa, bn=1000):
    n, d = x.shape
    f = wl.shape[1]
    h = absa.shape[1]
    return pl.pallas_call(
        _proj_body,
        grid=(n // bn,),
        in_specs=[
            pl.BlockSpec((bn, d), lambda i: (i, 0)),
            pl.BlockSpec((d, f), lambda i: (0, 0)),
            pl.BlockSpec((1, f), lambda i: (0, 0)),
            pl.BlockSpec((d, f), lambda i: (0, 0)),
            pl.BlockSpec((1, f), lambda i: (0, 0)),
            pl.BlockSpec((f, h), lambda i: (0, 0)),
        ],
        out_specs=[
            pl.BlockSpec((bn, f), lambda i: (i, 0)),
            pl.BlockSpec((bn, f), lambda i: (i, 0)),
            pl.BlockSpec((bn, h), lambda i: (i, 0)),
        ],
        out_shape=[
            jax.ShapeDtypeStruct((n, f), jnp.float32),
            jax.ShapeDtypeStruct((n, f), jnp.float32),
            jax.ShapeDtypeStruct((n, h), jnp.float32),
        ],
    )(x, wl, bl, wr, br, absa)


def _colmax_body(p_ref, out_ref):
    out_ref[...] = jnp.max(p_ref[...], axis=0, keepdims=True)


def _colmax(p):
    _, h = p.shape
    return pl.pallas_call(
        _colmax_body,
        out_shape=jax.ShapeDtypeStruct((1, h), jnp.float32),
    )(p)


def _edge_body(gxl_ref, gxr_ref, maxp_ref, amask_ref, absa_ref, emaskt_ref,
               pick_ref, uw_ref, *, h, be, feff, fext):
    a = gxl_ref[...]
    b = gxr_ref[...]
    z = a + b
    lz = jnp.maximum(z, 0.2 * z)
    logits = jnp.dot(lz, amask_ref[...], precision=HIGH)
    sh = jnp.dot(jnp.abs(b), absa_ref[...], precision=HIGH) + maxp_ref[...]
    ex = jnp.exp(logits - sh)
    eid = pl.program_id(0) * be + lax.broadcasted_iota(jnp.int32, (be, 1), 0)
    ex = jnp.where(eid < A, ex, 0.0)
    exb = jnp.dot(ex, emaskt_ref[...], precision=HIGH)
    pad = fext - feff - h
    uw_ref[...] = jnp.concatenate(
        [a[:, :feff] * exb, ex, jnp.zeros((be, pad), jnp.float32)], axis=1)


def _edge(gxl, gxr, maxp, amask, absa, emaskt, pick, fext, be=1024):
    a_pad, f = gxl.shape
    h = amask.shape[1]
    feff = emaskt.shape[1]
    return pl.pallas_call(
        functools.partial(_edge_body, h=h, be=be, feff=feff, fext=fext),
        grid=(a_pad // be,),
        in_specs=[
            pl.BlockSpec((be, f), lambda i: (i, 0)),
            pl.BlockSpec((be, f), lambda i: (i, 0)),
            pl.BlockSpec((1, h), lambda i: (0, 0)),
            pl.BlockSpec((f, h), lambda i: (0, 0)),
            pl.BlockSpec((f, h), lambda i: (0, 0)),
            pl.BlockSpec((h, feff), lambda i: (0, 0)),
            pl.BlockSpec((feff, h), lambda i: (0, 0)),
        ],
        out_specs=pl.BlockSpec((be, fext), lambda i: (i, 0)),
        out_shape=jax.ShapeDtypeStruct((a_pad, fext), jnp.float32),
    )(gxl, gxr, maxp, amask, absa, emaskt, pick)


def _mid_body(uo_ref, b1_ref, emaskt_ref, wl_ref, bl_ref,
              wr_ref, br_ref, absa_ref, xl_ref, xr_ref, p_ref):
    u = uo_ref[0, :, 0:512] + uo_ref[1, :, 0:512]
    d8 = uo_ref[0, :, 512:520] + uo_ref[1, :, 512:520]
    db = jnp.dot(d8, emaskt_ref[...], precision=HIGH) + 1e-16
    hid = u / db + b1_ref[...]
    hid = jnp.where(hid > 0, hid, jnp.exp(jnp.minimum(hid, 0.0)) - 1.0)
    xl = jnp.dot(hid, wl_ref[...], precision=HIGH) + bl_ref[...]
    xr = jnp.dot(hid, wr_ref[...], precision=HIGH) + br_ref[...]
    pad = jnp.zeros((xl.shape[0], 128 - xl.shape[1]), jnp.float32)
    xl_ref[...] = jnp.concatenate([xl, pad], axis=1)
    xr_ref[...] = jnp.concatenate([xr, pad], axis=1)
    p_ref[...] = jnp.dot(jnp.abs(xl), absa_ref[...], precision=HIGH)


def _mid(uo, b1, emaskt, wl2, bl2, wr2, br2, absa2, bn=1024):
    f = uo.shape[2]
    f2 = wl2.shape[1]
    h2 = absa2.shape[1]
    return pl.pallas_call(
        _mid_body,
        grid=(NP // bn,),
        in_specs=[
            pl.BlockSpec((2, bn, f), lambda i: (0, i, 0)),
            pl.BlockSpec((1, 512), lambda i: (0, 0)),
            pl.BlockSpec((8, 512), lambda i: (0, 0)),
            pl.BlockSpec((512, f2), lambda i: (0, 0)),
            pl.BlockSpec((1, f2), lambda i: (0, 0)),
            pl.BlockSpec((512, f2), lambda i: (0, 0)),
            pl.BlockSpec((1, f2), lambda i: (0, 0)),
            pl.BlockSpec((f2, h2), lambda i: (0, 0)),
        ],
        out_specs=[
            pl.BlockSpec((bn, 128), lambda i: (i, 0)),
            pl.BlockSpec((bn, 128), lambda i: (i, 0)),
            pl.BlockSpec((bn, h2), lambda i: (i, 0)),
        ],
        out_shape=[
            jax.ShapeDtypeStruct((NP, 128), jnp.float32),
            jax.ShapeDtypeStruct((NP, 128), jnp.float32),
            jax.ShapeDtypeStruct((NP, h2), jnp.float32),
        ],
    )(uo, b1, emaskt, wl2, bl2, wr2, br2, absa2)


def _fin_body(uo_ref, b2_ref, wlin_ref, blin_ref, out_ref):
    u = uo_ref[0, :, 0:64] + uo_ref[1, :, 0:64]
    d = uo_ref[0, :, 64:65] + uo_ref[1, :, 64:65]
    hid = u / (d + 1e-16) + b2_ref[...]
    hid = jnp.where(hid > 0, hid, jnp.exp(jnp.minimum(hid, 0.0)) - 1.0)
    out_ref[...] = jnp.dot(hid, wlin_ref[...], precision=HIGH) + blin_ref[...]


def _fin(uo, b2, wlin, blin, bn=1024):
    f = uo.shape[2]
    fo = wlin.shape[1]
    return pl.pallas_call(
        _fin_body,
        grid=(NP // bn,),
        in_specs=[
            pl.BlockSpec((2, bn, f), lambda i: (0, i, 0)),
            pl.BlockSpec((1, 64), lambda i: (0, 0)),
            pl.BlockSpec((64, fo), lambda i: (0, 0)),
            pl.BlockSpec((1, fo), lambda i: (0, 0)),
        ],
        out_specs=pl.BlockSpec((bn, fo), lambda i: (i, 0)),
        out_shape=jax.ShapeDtypeStruct((NP, fo), jnp.float32),
    )(uo, b2, wlin, blin)


def _gather(xl, xr, src, dst, c2=40):
    f = xl.shape[1]
    nch = EW // c2
    npairs = nch // 2

    @functools.partial(
        pl.kernel,
        mesh=_mesh(),
        out_type=[
            jax.ShapeDtypeStruct((A_PAD, f), jnp.float32),
            jax.ShapeDtypeStruct((A_PAD, f), jnp.float32),
        ],
        scratch_types=[
            pltpu.VMEM((c2,), jnp.int32),
            pltpu.VMEM((c2,), jnp.int32),
            pltpu.VMEM((c2,), jnp.int32),
            pltpu.VMEM((c2,), jnp.int32),
            pltpu.VMEM((c2, f), jnp.float32),
            pltpu.VMEM((c2, f), jnp.float32),
            pltpu.VMEM((c2, f), jnp.float32),
            pltpu.VMEM((c2, f), jnp.float32),
        ] + [pltpu.SemaphoreType.DMA] * 8,
    )
    def k(xl_h, xr_h, src_h, dst_h, gxl_h, gxr_h,
          si0, di0, si1, di1, bl0, br0, bl1, br1,
          sgl0, sgr0, sgl1, sgr1, swl0, swr0, swl1, swr1):
        wid = lax.axis_index("s") * 2 + lax.axis_index("c")
        base = wid * EW

        def wait_g(bl, br, sgl, sgr):
            pltpu.make_async_copy(xl_h.at[pl.ds(0, c2)], bl, sgl).wait()
            pltpu.make_async_copy(xr_h.at[pl.ds(0, c2)], br, sgr).wait()

        def wait_w(bl, br, swl, swr):
            pltpu.make_async_copy(bl, gxl_h.at[pl.ds(0, c2)], swl).wait()
            pltpu.make_async_copy(br, gxr_h.at[pl.ds(0, c2)], swr).wait()

        pltpu.sync_copy(src_h.at[pl.ds(base, c2)], si0)
        pltpu.sync_copy(dst_h.at[pl.ds(base, c2)], di0)
        pltpu.async_copy(xl_h.at[si0], bl0, sgl0)
        pltpu.async_copy(xr_h.at[di0], br0, sgr0)

        @pl.loop(0, npairs)
        def _(p):
            c0 = base + (2 * p) * c2
            c1 = c0 + c2
            cn = c0 + 2 * c2
            wait_g(bl0, br0, sgl0, sgr0)
            pltpu.async_copy(bl0, gxl_h.at[pl.ds(c0, c2)], swl0)
            pltpu.async_copy(br0, gxr_h.at[pl.ds(c0, c2)], swr0)

            pltpu.sync_copy(src_h.at[pl.ds(c1, c2)], si1)
            pltpu.sync_copy(dst_h.at[pl.ds(c1, c2)], di1)

            @pl.when(p > 0)
            def _():
                wait_w(bl1, br1, swl1, swr1)

            pltpu.async_copy(xl_h.at[si1], bl1, sgl1)
            pltpu.async_copy(xr_h.at[di1], br1, sgr1)
            wait_g(bl1, br1, sgl1, sgr1)
            pltpu.async_copy(bl1, gxl_h.at[pl.ds(c1, c2)], swl1)
            pltpu.async_copy(br1, gxr_h.at[pl.ds(c1, c2)], swr1)

            @pl.when(p < npairs - 1)
            def _():
                pltpu.sync_copy(src_h.at[pl.ds(cn, c2)], si0)
                pltpu.sync_copy(dst_h.at[pl.ds(cn, c2)], di0)
                wait_w(bl0, br0, swl0, swr0)
                pltpu.async_copy(xl_h.at[si0], bl0, sgl0)
                pltpu.async_copy(xr_h.at[di0], br0, sgr0)

        wait_w(bl0, br0, swl0, swr0)
        wait_w(bl1, br1, swl1, swr1)

    return k(xl, xr, src, dst)


def _segsum(uw, dst, rowidx, z80, slices, c4=40):
    f = uw.shape[1]
    cw = f // slices
    nch = EW // c4
    nzch = ROWS_W // c4

    @functools.partial(
        pl.kernel,
        mesh=_mesh(),
        out_type=jax.ShapeDtypeStruct((2, NP, f), jnp.float32),
        scratch_types=[
            pltpu.VMEM_SHARED((NP, cw), jnp.float32),
            pltpu.VMEM((c4, cw), jnp.float32),
            pltpu.VMEM((c4, cw), jnp.float32),
            pltpu.VMEM((c4,), jnp.int32),
            pltpu.VMEM((c4,), jnp.int32),
            pltpu.VMEM((c4,), jnp.int32),
        ] + [pltpu.SemaphoreType.DMA] * 6,
    )
    def k(uw_h, dst_h, ri_h, z80_h, uo_h, slabm, bufm, bufm2, ibuf, ibuf2,
          ribuf, sli0, slm0, sli1, slm1, ssc0, ssc1):
        c = lax.axis_index("c")
        s = lax.axis_index("s")
        wid = s * 2 + c
        base = wid * EW
        rs = s * ROWS_W

        for sl in range(slices):
            pltpu.sync_copy(z80_h, bufm)

            @pl.loop(0, nzch)
            def _(r):
                pltpu.sync_copy(ri_h.at[pl.ds(rs + r * c4, c4)], ribuf)
                pltpu.sync_copy(bufm, slabm.at[ribuf])

            plsc.subcore_barrier()

            def load(i, ib, bm, sli, slm):
                off = base + i * c4
                pltpu.async_copy(dst_h.at[pl.ds(off, c4)], ib, sli)
                if slices == 1:
                    pltpu.async_copy(uw_h.at[pl.ds(off, c4)], bm, slm)
                else:
                    pltpu.async_copy(
                        uw_h.at[pl.ds(off, c4), pl.ds(sl * cw, cw)], bm, slm)

            def wait_load(ib, bm, sli, slm):
                pltpu.make_async_copy(dst_h.at[pl.ds(0, c4)], ib, sli).wait()
                pltpu.make_async_copy(
                    uw_h.at[pl.ds(0, c4), pl.ds(0, cw)], bm, slm).wait()

            def wait_sc(bm, ssc):
                pltpu.make_async_copy(bm, slabm.at[pl.ds(0, c4)], ssc).wait()

            load(0, ibuf, bufm, sli0, slm0)

            @pl.loop(0, nch // 2)
            def _(p):
                wait_load(ibuf, bufm, sli0, slm0)
                pltpu.async_copy(bufm, slabm.at[ibuf], ssc0, add=True)

                @pl.when(p > 0)
                def _():
                    wait_sc(bufm2, ssc1)

                load(2 * p + 1, ibuf2, bufm2, sli1, slm1)
                wait_load(ibuf2, bufm2, sli1, slm1)
                pltpu.async_copy(bufm2, slabm.at[ibuf2], ssc1, add=True)

                @pl.when(p < nch // 2 - 1)
                def _():
                    wait_sc(bufm, ssc0)
                    load(2 * p + 2, ibuf, bufm, sli0, slm0)

            wait_sc(bufm, ssc0)
            wait_sc(bufm2, ssc1)
            plsc.subcore_barrier()

            @pl.loop(0, nzch)
            def _(r):
                row = rs + r * c4
                pltpu.sync_copy(ri_h.at[pl.ds(row, c4)], ribuf)
                pltpu.sync_copy(slabm.at[ribuf], bufm)
                if slices == 1:
                    pltpu.sync_copy(bufm, uo_h.at[c, pl.ds(row, c4)])
                else:
                    pltpu.sync_copy(
                        bufm, uo_h.at[c, pl.ds(row, c4), pl.ds(sl * cw, cw)])

            plsc.subcore_barrier()

    return k(uw, dst, rowidx, z80)


def kernel(x, edge_index, Wl1, bl1, Wr1, br1, att1, b1,
           Wl2, bl2, Wr2, br2, att2, b2, Wlin, blin):
    f32 = jnp.float32
    loop = jnp.arange(N, dtype=jnp.int32)
    padz = jnp.zeros((A_PAD - A,), jnp.int32)
    src = jnp.concatenate([edge_index[0].astype(jnp.int32), loop, padz])
    dst = jnp.concatenate([edge_index[1].astype(jnp.int32), loop, padz])
    rowidx = jnp.arange(NP, dtype=jnp.int32)

    attf1 = att1.reshape(-1).astype(f32)
    hm1 = (jnp.arange(512)[:, None] // 64) == jnp.arange(8)[None, :]
    amask1 = jnp.where(hm1, attf1[:, None], 0.0)
    absa1 = jnp.where(hm1, jnp.abs(attf1)[:, None], 0.0)
    emaskt1 = hm1.astype(f32).T
    attf2 = att2.reshape(-1).astype(f32)
    amask2 = jnp.concatenate(
        [attf2[:, None], jnp.zeros((64, 1), f32)], axis=0)
    absa2 = jnp.abs(amask2)
    absa2u = jnp.abs(attf2)[:, None]
    emaskt2 = jnp.ones((1, 64), f32)

    pick1 = (jnp.arange(512)[:, None] == jnp.arange(8)[None, :] * 64
             ).astype(f32)
    pick2 = (jnp.arange(64)[:, None] == 0).astype(f32)

    z80 = jnp.zeros((40, 128), f32)

    xl1, xr1, p1 = _proj(x, Wl1, bl1.reshape(1, -1), Wr1,
                         br1.reshape(1, -1), absa1)
    maxp1 = _colmax(p1)
    gxl1, gxr1 = _gather(xl1, xr1, src, dst)
    uw1 = _edge(gxl1, gxr1, maxp1, amask1, absa1, emaskt1, pick1, fext=640)
    uo1 = _segsum(uw1, dst, rowidx, z80, slices=5)

    xl2, xr2, p2 = _mid(uo1, b1.reshape(1, -1), emaskt1,
                        Wl2, bl2.reshape(1, -1), Wr2,
                        br2.reshape(1, -1), absa2u)
    maxp2 = _colmax(p2)
    gxl2, gxr2 = _gather(xl2, xr2, src, dst)
    uw2 = _edge(gxl2, gxr2, maxp2, amask2, absa2, emaskt2, pick2, fext=128)
    uo2 = _segsum(uw2, dst, rowidx, z80, slices=1)

    out = _fin(uo2, b2.reshape(1, -1), Wlin, blin.reshape(1, -1))
    return out[:N]


# speedup vs baseline: 1.3691x; 1.0004x over previous
import functools

import jax
import jax.numpy as jnp
from jax import lax
from jax.experimental import pallas as pl
from jax.experimental.pallas import tpu as pltpu
from jax.experimental.pallas import tpu_sc as plsc

HIGH = lax.Precision.HIGHEST

N = 10000
E = 320000
A = E + N
NW = 32
EW = 10320
A_PAD = EW * NW
NP = 10240
ROWS_W = NP // 16

_MESH = None


def _mesh():
    global _MESH
    if _MESH is None:
        _MESH = plsc.VectorSubcoreMesh(core_axis_name="c", subcore_axis_name="s")
    return _MESH


def _proj_body(x_ref, wl_ref, bl_ref, wr_ref, br_ref, absa_ref,
               xl_ref, xr_ref, p_ref):
    xb = x_ref[...]
    xl = jnp.dot(xb, wl_ref[...], precision=HIGH) + bl_ref[...]
    xr = jnp.dot(xb, wr_ref[...], precision=HIGH) + br_ref[...]
    xl_ref[...] = xl
    xr_ref[...] = xr
    p_ref[...] = jnp.dot(jnp.abs(xl), absa_ref[...], precision=HIGH)


def _proj(x, wl, bl, wr, br, absa, bn=1000):
    n, d = x.shape
    f = wl.shape[1]
    h = absa.shape[1]
    return pl.pallas_call(
        _proj_body,
        grid=(n // bn,),
        in_specs=[
            pl.BlockSpec((bn, d), lambda i: (i, 0)),
            pl.BlockSpec((d, f), lambda i: (0, 0)),
            pl.BlockSpec((1, f), lambda i: (0, 0)),
            pl.BlockSpec((d, f), lambda i: (0, 0)),
            pl.BlockSpec((1, f), lambda i: (0, 0)),
            pl.BlockSpec((f, h), lambda i: (0, 0)),
        ],
        out_specs=[
            pl.BlockSpec((bn, f), lambda i: (i, 0)),
            pl.BlockSpec((bn, f), lambda i: (i, 0)),
            pl.BlockSpec((bn, h), lambda i: (i, 0)),
        ],
        out_shape=[
            jax.ShapeDtypeStruct((n, f), jnp.float32),
            jax.ShapeDtypeStruct((n, f), jnp.float32),
            jax.ShapeDtypeStruct((n, h), jnp.float32),
        ],
    )(x, wl, bl, wr, br, absa)


def _colmax_body(p_ref, out_ref):
    out_ref[...] = jnp.max(p_ref[...], axis=0, keepdims=True)


def _colmax(p):
    _, h = p.shape
    return pl.pallas_call(
        _colmax_body,
        out_shape=jax.ShapeDtypeStruct((1, h), jnp.float32),
    )(p)


def _edge_body(gxl_ref, gxr_ref, maxp_ref, amask_ref, absa_ref, emaskt_ref,
               pick_ref, uw_ref, *, h, be, feff, fext):
    a = gxl_ref[...]
    b = gxr_ref[...]
    z = a + b
    lz = jnp.maximum(z, 0.2 * z)
    logits = jnp.dot(lz, amask_ref[...], precision=HIGH)
    sh = jnp.dot(jnp.abs(b), absa_ref[...], precision=HIGH) + maxp_ref[...]
    ex = jnp.exp(logits - sh)
    eid = pl.program_id(0) * be + lax.broadcasted_iota(jnp.int32, (be, 1), 0)
    ex = jnp.where(eid < A, ex, 0.0)
    exb = jnp.dot(ex, emaskt_ref[...], precision=HIGH)
    pad = fext - feff - h
    uw_ref[...] = jnp.concatenate(
        [a[:, :feff] * exb, ex, jnp.zeros((be, pad), jnp.float32)], axis=1)


def _edge(gxl, gxr, maxp, amask, absa, emaskt, pick, fext, be=960):
    a_pad, f = gxl.shape
    h = amask.shape[1]
    feff = emaskt.shape[1]
    return pl.pallas_call(
        functools.partial(_edge_body, h=h, be=be, feff=feff, fext=fext),
        grid=(a_pad // be,),
        in_specs=[
            pl.BlockSpec((be, f), lambda i: (i, 0)),
            pl.BlockSpec((be, f), lambda i: (i, 0)),
            pl.BlockSpec((1, h), lambda i: (0, 0)),
            pl.BlockSpec((f, h), lambda i: (0, 0)),
            pl.BlockSpec((f, h), lambda i: (0, 0)),
            pl.BlockSpec((h, feff), lambda i: (0, 0)),
            pl.BlockSpec((feff, h), lambda i: (0, 0)),
        ],
        out_specs=pl.BlockSpec((be, fext), lambda i: (i, 0)),
        out_shape=jax.ShapeDtypeStruct((a_pad, fext), jnp.float32),
    )(gxl, gxr, maxp, amask, absa, emaskt, pick)


def _mid_body(uo_ref, b1_ref, emaskt_ref, wl_ref, bl_ref,
              wr_ref, br_ref, absa_ref, xl_ref, xr_ref, p_ref):
    u = uo_ref[0, :, 0:512] + uo_ref[1, :, 0:512]
    d8 = uo_ref[0, :, 512:520] + uo_ref[1, :, 512:520]
    db = jnp.dot(d8, emaskt_ref[...], precision=HIGH) + 1e-16
    hid = u / db + b1_ref[...]
    hid = jnp.where(hid > 0, hid, jnp.exp(jnp.minimum(hid, 0.0)) - 1.0)
    xl = jnp.dot(hid, wl_ref[...], precision=HIGH) + bl_ref[...]
    xr = jnp.dot(hid, wr_ref[...], precision=HIGH) + br_ref[...]
    pad = jnp.zeros((xl.shape[0], 128 - xl.shape[1]), jnp.float32)
    xl_ref[...] = jnp.concatenate([xl, pad], axis=1)
    xr_ref[...] = jnp.concatenate([xr, pad], axis=1)
    p_ref[...] = jnp.dot(jnp.abs(xl), absa_ref[...], precision=HIGH)


def _mid(uo, b1, emaskt, wl2, bl2, wr2, br2, absa2, bn=1024):
    f = uo.shape[2]
    f2 = wl2.shape[1]
    h2 = absa2.shape[1]
    return pl.pallas_call(
        _mid_body,
        grid=(NP // bn,),
        in_specs=[
            pl.BlockSpec((2, bn, f), lambda i: (0, i, 0)),
            pl.BlockSpec((1, 512), lambda i: (0, 0)),
            pl.BlockSpec((8, 512), lambda i: (0, 0)),
            pl.BlockSpec((512, f2), lambda i: (0, 0)),
            pl.BlockSpec((1, f2), lambda i: (0, 0)),
            pl.BlockSpec((512, f2), lambda i: (0, 0)),
            pl.BlockSpec((1, f2), lambda i: (0, 0)),
            pl.BlockSpec((f2, h2), lambda i: (0, 0)),
        ],
        out_specs=[
            pl.BlockSpec((bn, 128), lambda i: (i, 0)),
            pl.BlockSpec((bn, 128), lambda i: (i, 0)),
            pl.BlockSpec((bn, h2), lambda i: (i, 0)),
        ],
        out_shape=[
            jax.ShapeDtypeStruct((NP, 128), jnp.float32),
            jax.ShapeDtypeStruct((NP, 128), jnp.float32),
            jax.ShapeDtypeStruct((NP, h2), jnp.float32),
        ],
    )(uo, b1, emaskt, wl2, bl2, wr2, br2, absa2)


def _fin_body(uo_ref, b2_ref, wlin_ref, blin_ref, out_ref):
    u = uo_ref[0, :, 0:64] + uo_ref[1, :, 0:64]
    d = uo_ref[0, :, 64:65] + uo_ref[1, :, 64:65]
    hid = u / (d + 1e-16) + b2_ref[...]
    hid = jnp.where(hid > 0, hid, jnp.exp(jnp.minimum(hid, 0.0)) - 1.0)
    out_ref[...] = jnp.dot(hid, wlin_ref[...], precision=HIGH) + blin_ref[...]


def _fin(uo, b2, wlin, blin, bn=1024):
    f = uo.shape[2]
    fo = wlin.shape[1]
    return pl.pallas_call(
        _fin_body,
        grid=(NP // bn,),
        in_specs=[
            pl.BlockSpec((2, bn, f), lambda i: (0, i, 0)),
            pl.BlockSpec((1, 64), lambda i: (0, 0)),
            pl.BlockSpec((64, fo), lambda i: (0, 0)),
            pl.BlockSpec((1, fo), lambda i: (0, 0)),
        ],
        out_specs=pl.BlockSpec((bn, fo), lambda i: (i, 0)),
        out_shape=jax.ShapeDtypeStruct((NP, fo), jnp.float32),
    )(uo, b2, wlin, blin)


def _gather(xl, xr, src, dst, c2=40):
    f = xl.shape[1]
    nch = EW // c2
    npairs = nch // 2

    @functools.partial(
        pl.kernel,
        mesh=_mesh(),
        out_type=[
            jax.ShapeDtypeStruct((A_PAD, f), jnp.float32),
            jax.ShapeDtypeStruct((A_PAD, f), jnp.float32),
        ],
        scratch_types=[
            pltpu.VMEM((c2,), jnp.int32),
            pltpu.VMEM((c2,), jnp.int32),
            pltpu.VMEM((c2,), jnp.int32),
            pltpu.VMEM((c2,), jnp.int32),
            pltpu.VMEM((c2, f), jnp.float32),
            pltpu.VMEM((c2, f), jnp.float32),
            pltpu.VMEM((c2, f), jnp.float32),
            pltpu.VMEM((c2, f), jnp.float32),
        ] + [pltpu.SemaphoreType.DMA] * 8,
    )
    def k(xl_h, xr_h, src_h, dst_h, gxl_h, gxr_h,
          si0, di0, si1, di1, bl0, br0, bl1, br1,
          sgl0, sgr0, sgl1, sgr1, swl0, swr0, swl1, swr1):
        wid = lax.axis_index("s") * 2 + lax.axis_index("c")
        base = wid * EW

        def wait_g(bl, br, sgl, sgr):
            pltpu.make_async_copy(xl_h.at[pl.ds(0, c2)], bl, sgl).wait()
            pltpu.make_async_copy(xr_h.at[pl.ds(0, c2)], br, sgr).wait()

        def wait_w(bl, br, swl, swr):
            pltpu.make_async_copy(bl, gxl_h.at[pl.ds(0, c2)], swl).wait()
            pltpu.make_async_copy(br, gxr_h.at[pl.ds(0, c2)], swr).wait()

        pltpu.sync_copy(src_h.at[pl.ds(base, c2)], si0)
        pltpu.sync_copy(dst_h.at[pl.ds(base, c2)], di0)
        pltpu.async_copy(xl_h.at[si0], bl0, sgl0)
        pltpu.async_copy(xr_h.at[di0], br0, sgr0)

        @pl.loop(0, npairs)
        def _(p):
            c0 = base + (2 * p) * c2
            c1 = c0 + c2
            cn = c0 + 2 * c2
            wait_g(bl0, br0, sgl0, sgr0)
            pltpu.async_copy(bl0, gxl_h.at[pl.ds(c0, c2)], swl0)
            pltpu.async_copy(br0, gxr_h.at[pl.ds(c0, c2)], swr0)

            pltpu.sync_copy(src_h.at[pl.ds(c1, c2)], si1)
            pltpu.sync_copy(dst_h.at[pl.ds(c1, c2)], di1)

            @pl.when(p > 0)
            def _():
                wait_w(bl1, br1, swl1, swr1)

            pltpu.async_copy(xl_h.at[si1], bl1, sgl1)
            pltpu.async_copy(xr_h.at[di1], br1, sgr1)
            wait_g(bl1, br1, sgl1, sgr1)
            pltpu.async_copy(bl1, gxl_h.at[pl.ds(c1, c2)], swl1)
            pltpu.async_copy(br1, gxr_h.at[pl.ds(c1, c2)], swr1)

            @pl.when(p < npairs - 1)
            def _():
                pltpu.sync_copy(src_h.at[pl.ds(cn, c2)], si0)
                pltpu.sync_copy(dst_h.at[pl.ds(cn, c2)], di0)
                wait_w(bl0, br0, swl0, swr0)
                pltpu.async_copy(xl_h.at[si0], bl0, sgl0)
                pltpu.async_copy(xr_h.at[di0], br0, sgr0)

        wait_w(bl0, br0, swl0, swr0)
        wait_w(bl1, br1, swl1, swr1)

    return k(xl, xr, src, dst)


def _segsum(uw, dst, rowidx, z80, slices, c4=40):
    f = uw.shape[1]
    cw = f // slices
    nch = EW // c4
    nzch = ROWS_W // c4

    @functools.partial(
        pl.kernel,
        mesh=_mesh(),
        out_type=jax.ShapeDtypeStruct((2, NP, f), jnp.float32),
        scratch_types=[
            pltpu.VMEM_SHARED((NP, cw), jnp.float32),
            pltpu.VMEM((c4, cw), jnp.float32),
            pltpu.VMEM((c4, cw), jnp.float32),
            pltpu.VMEM((c4,), jnp.int32),
            pltpu.VMEM((c4,), jnp.int32),
            pltpu.VMEM((c4,), jnp.int32),
        ] + [pltpu.SemaphoreType.DMA] * 6,
    )
    def k(uw_h, dst_h, ri_h, z80_h, uo_h, slabm, bufm, bufm2, ibuf, ibuf2,
          ribuf, sli0, slm0, sli1, slm1, ssc0, ssc1):
        c = lax.axis_index("c")
        s = lax.axis_index("s")
        wid = s * 2 + c
        base = wid * EW
        rs = s * ROWS_W

        for sl in range(slices):
            pltpu.sync_copy(z80_h, bufm)

            @pl.loop(0, nzch)
            def _(r):
                pltpu.sync_copy(ri_h.at[pl.ds(rs + r * c4, c4)], ribuf)
                pltpu.sync_copy(bufm, slabm.at[ribuf])

            plsc.subcore_barrier()

            def load(i, ib, bm, sli, slm):
                off = base + i * c4
                pltpu.async_copy(dst_h.at[pl.ds(off, c4)], ib, sli)
                if slices == 1:
                    pltpu.async_copy(uw_h.at[pl.ds(off, c4)], bm, slm)
                else:
                    pltpu.async_copy(
                        uw_h.at[pl.ds(off, c4), pl.ds(sl * cw, cw)], bm, slm)

            def wait_load(ib, bm, sli, slm):
                pltpu.make_async_copy(dst_h.at[pl.ds(0, c4)], ib, sli).wait()
                pltpu.make_async_copy(
                    uw_h.at[pl.ds(0, c4), pl.ds(0, cw)], bm, slm).wait()

            def wait_sc(bm, ssc):
                pltpu.make_async_copy(bm, slabm.at[pl.ds(0, c4)], ssc).wait()

            load(0, ibuf, bufm, sli0, slm0)

            @pl.loop(0, nch // 2)
            def _(p):
                wait_load(ibuf, bufm, sli0, slm0)
                pltpu.async_copy(bufm, slabm.at[ibuf], ssc0, add=True)

                @pl.when(p > 0)
                def _():
                    wait_sc(bufm2, ssc1)

                load(2 * p + 1, ibuf2, bufm2, sli1, slm1)
                wait_load(ibuf2, bufm2, sli1, slm1)
                pltpu.async_copy(bufm2, slabm.at[ibuf2], ssc1, add=True)

                @pl.when(p < nch // 2 - 1)
                def _():
                    wait_sc(bufm, ssc0)
                    load(2 * p + 2, ibuf, bufm, sli0, slm0)

            wait_sc(bufm, ssc0)
            wait_sc(bufm2, ssc1)
            plsc.subcore_barrier()

            @pl.loop(0, nzch)
            def _(r):
                row = rs + r * c4
                pltpu.sync_copy(ri_h.at[pl.ds(row, c4)], ribuf)
                pltpu.sync_copy(slabm.at[ribuf], bufm)
                if slices == 1:
                    pltpu.sync_copy(bufm, uo_h.at[c, pl.ds(row, c4)])
                else:
                    pltpu.sync_copy(
                        bufm, uo_h.at[c, pl.ds(row, c4), pl.ds(sl * cw, cw)])

            plsc.subcore_barrier()

    return k(uw, dst, rowidx, z80)


def kernel(x, edge_index, Wl1, bl1, Wr1, br1, att1, b1,
           Wl2, bl2, Wr2, br2, att2, b2, Wlin, blin):
    f32 = jnp.float32
    loop = jnp.arange(N, dtype=jnp.int32)
    padz = jnp.zeros((A_PAD - A,), jnp.int32)
    src = jnp.concatenate([edge_index[0].astype(jnp.int32), loop, padz])
    dst = jnp.concatenate([edge_index[1].astype(jnp.int32), loop, padz])
    rowidx = jnp.arange(NP, dtype=jnp.int32)

    attf1 = att1.reshape(-1).astype(f32)
    hm1 = (jnp.arange(512)[:, None] // 64) == jnp.arange(8)[None, :]
    amask1 = jnp.where(hm1, attf1[:, None], 0.0)
    absa1 = jnp.where(hm1, jnp.abs(attf1)[:, None], 0.0)
    emaskt1 = hm1.astype(f32).T
    attf2 = att2.reshape(-1).astype(f32)
    amask2 = jnp.concatenate(
        [attf2[:, None], jnp.zeros((64, 1), f32)], axis=0)
    absa2 = jnp.abs(amask2)
    absa2u = jnp.abs(attf2)[:, None]
    emaskt2 = jnp.ones((1, 64), f32)

    pick1 = (jnp.arange(512)[:, None] == jnp.arange(8)[None, :] * 64
             ).astype(f32)
    pick2 = (jnp.arange(64)[:, None] == 0).astype(f32)

    z80 = jnp.zeros((40, 128), f32)

    xl1, xr1, p1 = _proj(x, Wl1, bl1.reshape(1, -1), Wr1,
                         br1.reshape(1, -1), absa1)
    maxp1 = _colmax(p1)
    gxl1, gxr1 = _gather(xl1, xr1, src, dst)
    uw1 = _edge(gxl1, gxr1, maxp1, amask1, absa1, emaskt1, pick1, fext=640)
    uo1 = _segsum(uw1, dst, rowidx, z80, slices=5)

    xl2, xr2, p2 = _mid(uo1, b1.reshape(1, -1), emaskt1,
                        Wl2, bl2.reshape(1, -1), Wr2,
                        br2.reshape(1, -1), absa2u)
    maxp2 = _colmax(p2)
    gxl2, gxr2 = _gather(xl2, xr2, src, dst)
    uw2 = _edge(gxl2, gxr2, maxp2, amask2, absa2, emaskt2, pick2, fext=128)
    uo2 = _segsum(uw2, dst, rowidx, z80, slices=1)

    out = _fin(uo2, b2.reshape(1, -1), Wlin, blin.reshape(1, -1))
    return out[:N]
